# baseline (speedup 1.0000x reference)
import functools

import jax
import jax.numpy as jnp
from jax.experimental import pallas as pl


def _fused_kernel(x_ref, w_ref, o_ref):
    x = x_ref[:]
    s = jnp.maximum(jnp.sum(x, axis=1, keepdims=True), 1.0)
    y = jnp.dot(x, w_ref[:], preferred_element_type=jnp.float32)
    o_ref[:] = y / s


@functools.partial(jax.jit, static_argnames=("block_rows",))
def _run(x2d, W, block_rows):
    rows, vocab = x2d.shape
    dim = W.shape[1]
    grid = (rows // block_rows,)
    return pl.pallas_call(
        _fused_kernel,
        grid=grid,
        in_specs=[
            pl.BlockSpec((block_rows, vocab), lambda i: (i, 0)),
            pl.BlockSpec((vocab, dim), lambda i: (0, 0)),
        ],
        out_specs=pl.BlockSpec((block_rows, dim), lambda i: (i, 0)),
        out_shape=jax.ShapeDtypeStruct((rows, dim), jnp.float32),
    )(x2d, W)


def kernel(x_multi_hot, W):
    b, t, vocab = x_multi_hot.shape
    x2d = x_multi_hot.reshape(b * t, vocab)
    out = _run(x2d, W, 512)
    return out.reshape(b, t, W.shape[1])


# speedup vs baseline: 1.6034x; 1.6034x over previous
import functools

import jax
import jax.numpy as jnp
from jax.experimental import pallas as pl


def _fused_kernel(x_ref, w_ref, o_ref):
    x = x_ref[:]
    s = jnp.maximum(jnp.sum(x, axis=-1, keepdims=True), 1.0)
    y = jax.lax.dot_general(
        x, w_ref[:],
        dimension_numbers=(((2,), (0,)), ((), ())),
        preferred_element_type=jnp.float32,
    )
    o_ref[:] = y / s


@functools.partial(jax.jit, static_argnames=("block_b",))
def _run(x, W, block_b):
    b, t, vocab = x.shape
    dim = W.shape[1]
    grid = (b // block_b,)
    return pl.pallas_call(
        _fused_kernel,
        grid=grid,
        in_specs=[
            pl.BlockSpec((block_b, t, vocab), lambda i: (i, 0, 0)),
            pl.BlockSpec((vocab, dim), lambda i: (0, 0)),
        ],
        out_specs=pl.BlockSpec((block_b, t, dim), lambda i: (i, 0, 0)),
        out_shape=jax.ShapeDtypeStruct((b, t, dim), jnp.float32),
    )(x, W)


def kernel(x_multi_hot, W):
    return _run(x_multi_hot, W, 32)


# speedup vs baseline: 1.7416x; 1.0862x over previous
import functools

import jax
import jax.numpy as jnp
from jax.experimental import pallas as pl


def _fused_kernel(x_ref, w_ref, o_ref):
    x = x_ref[:]
    s = jnp.maximum(jnp.sum(x, axis=-1, keepdims=True), 1.0)
    y = jax.lax.dot_general(
        x, w_ref[:],
        dimension_numbers=(((2,), (0,)), ((), ())),
        preferred_element_type=jnp.float32,
    )
    o_ref[:] = y / s


@functools.partial(jax.jit, static_argnames=("block_b",))
def _run(x, W, block_b):
    b, t, vocab = x.shape
    dim = W.shape[1]
    grid = (b // block_b,)
    return pl.pallas_call(
        _fused_kernel,
        grid=grid,
        in_specs=[
            pl.BlockSpec((block_b, t, vocab), lambda i: (i, 0, 0)),
            pl.BlockSpec((vocab, dim), lambda i: (0, 0)),
        ],
        out_specs=pl.BlockSpec((block_b, t, dim), lambda i: (i, 0, 0)),
        out_shape=jax.ShapeDtypeStruct((b, t, dim), jnp.float32),
    )(x, W)


def kernel(x_multi_hot, W):
    return _run(x_multi_hot, W, 128)
